# baseline (speedup 1.0000x reference)
import jax
import jax.numpy as jnp
from jax.experimental import pallas as pl

_NPOINTS = [2048, 512, 128, 32]
_RADII = [0.1, 0.2, 0.4, 0.8]
_NSAMPLE = 32
_SA_CH = [[3, 32, 32, 64], [67, 64, 64, 128], [131, 128, 128, 256], [259, 256, 256, 512]]
_FP_CH = [[128, 128, 128, 128], [320, 256, 128], [384, 256, 256], [768, 256, 256]]


def _sqdist(a, b):
    return jnp.sum(a * a, -1)[:, :, None] + jnp.sum(b * b, -1)[:, None, :] - 2.0 * jnp.einsum('bnc,bmc->bnm', a, b)


def _gather(x, idx):
    return jax.vmap(lambda a, i: a[i])(x, idx)


def _fps(xyz, npoint):
    B, N, _ = xyz.shape
    def step(carry, _):
        dist, last = carry
        lx = jnp.take_along_axis(xyz, jnp.broadcast_to(last[:, None, None], (B, 1, 3)), axis=1)
        d = jnp.sum((xyz - lx) ** 2, axis=-1)
        dist = jnp.minimum(dist, d)
        nxt = jnp.argmax(dist, axis=1).astype(jnp.int32)
        return (dist, nxt), nxt
    first = jnp.zeros((B,), jnp.int32)
    (_, _), rest = jax.lax.scan(step, (jnp.full((B, N), 1e10, dtype=jnp.float32), first), None, length=npoint - 1)
    return jnp.concatenate([first[None, :], rest], axis=0).T


def _ball_query(radius, nsample, xyz, new_xyz):
    B, N, _ = xyz.shape
    d = _sqdist(new_xyz, xyz)
    mask = d < radius * radius
    ar = jnp.arange(N, dtype=jnp.int32)
    vals = jnp.where(mask, -ar, -(N + 1))
    tv, _ = jax.lax.top_k(vals, nsample)
    idx = -tv
    idx = jnp.where(idx < N, idx, idx[..., 0:1])
    return idx


def _sa(xyz, feats, npoint, radius, nsample, ws, bs):
    fidx = _fps(xyz, npoint)
    new_xyz = _gather(xyz, fidx)
    idx = _ball_query(radius, nsample, xyz, new_xyz)
    grouped = _gather(xyz, idx) - new_xyz[:, :, None, :]
    if feats is not None:
        grouped = jnp.concatenate([grouped, _gather(feats, idx)], axis=-1)
    x = grouped
    for W, b in zip(ws, bs):
        x = jax.nn.relu(jnp.einsum('bsnc,oc->bsno', x, W) + b)
    return new_xyz, jnp.max(x, axis=2)


def _fp(ux, kx, uf, kf, ws, bs):
    d = jnp.maximum(_sqdist(ux, kx), 0.0)
    negv, nidx = jax.lax.top_k(-d, 3)
    dist = -negv
    recip = 1.0 / (dist + 1e-8)
    w = recip / jnp.sum(recip, -1, keepdims=True)
    interp = jnp.sum(_gather(kf, nidx) * w[..., None], axis=2)
    x = interp if uf is None else jnp.concatenate([interp, uf], axis=-1)
    for W, b in zip(ws, bs):
        x = jax.nn.relu(jnp.einsum('bnc,oc->bno', x, W) + b)
    return x


def kernel(pointcloud, sa0_w0, sa0_b0, sa0_w1, sa0_b1, sa0_w2, sa0_b2, sa1_w0, sa1_b0, sa1_w1, sa1_b1, sa1_w2, sa1_b2, sa2_w0, sa2_b0, sa2_w1, sa2_b1, sa2_w2, sa2_b2, sa3_w0, sa3_b0, sa3_w1, sa3_b1, sa3_w2, sa3_b2, fp0_w0, fp0_b0, fp0_w1, fp0_b1, fp0_w2, fp0_b2, fp1_w0, fp1_b0, fp1_w1, fp1_b1, fp2_w0, fp2_b0, fp2_w1, fp2_b1, fp3_w0, fp3_b0, fp3_w1, fp3_b1):
    p = dict(locals())
    del p['pointcloud']
    xyz = pointcloud[..., 0:3]
    l_xyz = [xyz]
    l_f = [None]
    for L in range(4):
        ws = [p['sa%d_w%d' % (L, j)] for j in range(len(_SA_CH[L]) - 1)]
        bs = [p['sa%d_b%d' % (L, j)] for j in range(len(_SA_CH[L]) - 1)]
        nx, nf = _sa(l_xyz[L], l_f[L], _NPOINTS[L], _RADII[L], _NSAMPLE, ws, bs)
        l_xyz.append(nx)
        l_f.append(nf)
    for i in range(-1, -5, -1):
        L = 4 + i
        ws = [p['fp%d_w%d' % (L, j)] for j in range(len(_FP_CH[L]) - 1)]
        bs = [p['fp%d_b%d' % (L, j)] for j in range(len(_FP_CH[L]) - 1)]
        l_f[i - 1] = _fp(l_xyz[i - 1], l_xyz[i], l_f[i - 1], l_f[i], ws, bs)
    return jnp.transpose(l_f[0], (0, 2, 1))


# speedup vs baseline: 1.1567x; 1.1567x over previous
import jax
import jax.numpy as jnp
from jax.experimental import pallas as pl

_NPOINTS = [2048, 512, 128, 32]
_RADII = [0.1, 0.2, 0.4, 0.8]
_NSAMPLE = 32
_SA_CH = [[3, 32, 32, 64], [67, 64, 64, 128], [131, 128, 128, 256], [259, 256, 256, 512]]
_FP_CH = [[128, 128, 128, 128], [320, 256, 128], [384, 256, 256], [768, 256, 256]]


def _sqdist(a, b):
    return jnp.sum(a * a, -1)[:, :, None] + jnp.sum(b * b, -1)[:, None, :] - 2.0 * jnp.einsum('bnc,bmc->bnm', a, b)


def _gather(x, idx):
    return jax.vmap(lambda a, i: a[i])(x, idx)


def _fps(xyz, npoint):
    B, N, _ = xyz.shape
    C = 128
    R = N // C
    xs = jnp.transpose(xyz, (0, 2, 1)).reshape(B, 3, R, C)

    def kern(x_ref, o_ref):
        x = x_ref[0, 0]
        y = x_ref[0, 1]
        z = x_ref[0, 2]
        rr = jax.lax.broadcasted_iota(jnp.int32, (R, C), 0)
        cc = jax.lax.broadcasted_iota(jnp.int32, (R, C), 1)
        flat = rr * C + cc
        oio = jax.lax.broadcasted_iota(jnp.int32, (1, npoint), 1)
        o_ref[0] = jnp.zeros((1, npoint), jnp.int32)

        def body(k, carry):
            dist, last = carry
            oh = (flat == last).astype(jnp.float32)
            lx = jnp.sum(x * oh)
            ly = jnp.sum(y * oh)
            lz = jnp.sum(z * oh)
            d = (x - lx) ** 2 + (y - ly) ** 2 + (z - lz) ** 2
            dist = jnp.minimum(dist, d)
            m = jnp.max(dist)
            nxt = jnp.min(jnp.where(dist == m, flat, N)).astype(jnp.int32)
            o_ref[0] = jnp.where(oio == k, nxt, o_ref[0])
            return dist, nxt

        jax.lax.fori_loop(1, npoint, body,
                          (jnp.full((R, C), 1e10, jnp.float32), jnp.int32(0)))

    out = pl.pallas_call(
        kern,
        grid=(B,),
        in_specs=[pl.BlockSpec((1, 3, R, C), lambda b: (b, 0, 0, 0))],
        out_specs=pl.BlockSpec((1, 1, npoint), lambda b: (b, 0, 0)),
        out_shape=jax.ShapeDtypeStruct((B, 1, npoint), jnp.int32),
    )(xs)
    return out[:, 0, :]


def _ball_query(radius, nsample, xyz, new_xyz):
    B, N, _ = xyz.shape
    d = _sqdist(new_xyz, xyz)
    mask = d < radius * radius
    ar = jnp.arange(N, dtype=jnp.int32)
    vals = jnp.where(mask, -ar, -(N + 1))
    tv, _ = jax.lax.top_k(vals, nsample)
    idx = -tv
    idx = jnp.where(idx < N, idx, idx[..., 0:1])
    return idx


def _sa(xyz, feats, npoint, radius, nsample, ws, bs):
    fidx = _fps(xyz, npoint)
    new_xyz = _gather(xyz, fidx)
    idx = _ball_query(radius, nsample, xyz, new_xyz)
    grouped = _gather(xyz, idx) - new_xyz[:, :, None, :]
    if feats is not None:
        grouped = jnp.concatenate([grouped, _gather(feats, idx)], axis=-1)
    x = grouped
    for W, b in zip(ws, bs):
        x = jax.nn.relu(jnp.einsum('bsnc,oc->bsno', x, W) + b)
    return new_xyz, jnp.max(x, axis=2)


def _fp(ux, kx, uf, kf, ws, bs):
    d = jnp.maximum(_sqdist(ux, kx), 0.0)
    negv, nidx = jax.lax.top_k(-d, 3)
    dist = -negv
    recip = 1.0 / (dist + 1e-8)
    w = recip / jnp.sum(recip, -1, keepdims=True)
    interp = jnp.sum(_gather(kf, nidx) * w[..., None], axis=2)
    x = interp if uf is None else jnp.concatenate([interp, uf], axis=-1)
    for W, b in zip(ws, bs):
        x = jax.nn.relu(jnp.einsum('bnc,oc->bno', x, W) + b)
    return x


def kernel(pointcloud, sa0_w0, sa0_b0, sa0_w1, sa0_b1, sa0_w2, sa0_b2, sa1_w0, sa1_b0, sa1_w1, sa1_b1, sa1_w2, sa1_b2, sa2_w0, sa2_b0, sa2_w1, sa2_b1, sa2_w2, sa2_b2, sa3_w0, sa3_b0, sa3_w1, sa3_b1, sa3_w2, sa3_b2, fp0_w0, fp0_b0, fp0_w1, fp0_b1, fp0_w2, fp0_b2, fp1_w0, fp1_b0, fp1_w1, fp1_b1, fp2_w0, fp2_b0, fp2_w1, fp2_b1, fp3_w0, fp3_b0, fp3_w1, fp3_b1):
    p = dict(locals())
    del p['pointcloud']
    xyz = pointcloud[..., 0:3]
    l_xyz = [xyz]
    l_f = [None]
    for L in range(4):
        ws = [p['sa%d_w%d' % (L, j)] for j in range(len(_SA_CH[L]) - 1)]
        bs = [p['sa%d_b%d' % (L, j)] for j in range(len(_SA_CH[L]) - 1)]
        nx, nf = _sa(l_xyz[L], l_f[L], _NPOINTS[L], _RADII[L], _NSAMPLE, ws, bs)
        l_xyz.append(nx)
        l_f.append(nf)
    for i in range(-1, -5, -1):
        L = 4 + i
        ws = [p['fp%d_w%d' % (L, j)] for j in range(len(_FP_CH[L]) - 1)]
        bs = [p['fp%d_b%d' % (L, j)] for j in range(len(_FP_CH[L]) - 1)]
        l_f[i - 1] = _fp(l_xyz[i - 1], l_xyz[i], l_f[i - 1], l_f[i], ws, bs)
    return jnp.transpose(l_f[0], (0, 2, 1))


# speedup vs baseline: 1.1747x; 1.0155x over previous
import jax
import jax.numpy as jnp
from jax.experimental import pallas as pl

_NPOINTS = [2048, 512, 128, 32]
_RADII = [0.1, 0.2, 0.4, 0.8]
_NSAMPLE = 32
_SA_CH = [[3, 32, 32, 64], [67, 64, 64, 128], [131, 128, 128, 256], [259, 256, 256, 512]]
_FP_CH = [[128, 128, 128, 128], [320, 256, 128], [384, 256, 256], [768, 256, 256]]


def _sqdist(a, b):
    return jnp.sum(a * a, -1)[:, :, None] + jnp.sum(b * b, -1)[:, None, :] - 2.0 * jnp.einsum('bnc,bmc->bnm', a, b)


def _gather(x, idx):
    return jax.vmap(lambda a, i: a[i])(x, idx)


def _fps(xyz, npoint):
    B, N, _ = xyz.shape
    C = 128
    R = N // C
    xs = jnp.transpose(xyz, (0, 2, 1)).reshape(B, 3, R, C)

    def kern(x_ref, o_ref):
        rr = jax.lax.broadcasted_iota(jnp.int32, (R, C), 0)
        cc = jax.lax.broadcasted_iota(jnp.int32, (R, C), 1)
        flat = rr * C + cc
        oio = jax.lax.broadcasted_iota(jnp.int32, (1, npoint), 1)
        xb = []
        for b in range(B):
            o_ref[b] = jnp.zeros((1, npoint), jnp.int32)
            xb.append((x_ref[b, 0], x_ref[b, 1], x_ref[b, 2]))

        def body(k, carry):
            new = []
            for b in range(B):
                dist, last = carry[b]
                x, y, z = xb[b]
                oh = (flat == last).astype(jnp.float32)
                lx = jnp.sum(x * oh)
                ly = jnp.sum(y * oh)
                lz = jnp.sum(z * oh)
                d = (x - lx) ** 2 + (y - ly) ** 2 + (z - lz) ** 2
                dist = jnp.minimum(dist, d)
                m = jnp.max(dist)
                nxt = jnp.min(jnp.where(dist == m, flat, N)).astype(jnp.int32)
                o_ref[b] = jnp.where(oio == k, nxt, o_ref[b])
                new.append((dist, nxt))
            return tuple(new)

        jax.lax.fori_loop(
            1, npoint, body,
            tuple((jnp.full((R, C), 1e10, jnp.float32), jnp.int32(0))
                  for _ in range(B)))

    out = pl.pallas_call(
        kern,
        in_specs=[pl.BlockSpec((B, 3, R, C), lambda: (0, 0, 0, 0))],
        out_specs=pl.BlockSpec((B, 1, npoint), lambda: (0, 0, 0)),
        out_shape=jax.ShapeDtypeStruct((B, 1, npoint), jnp.int32),
    )(xs)
    return out[:, 0, :]


def _ball_query(radius, nsample, xyz, new_xyz):
    B, N, _ = xyz.shape
    d = _sqdist(new_xyz, xyz)
    mask = d < radius * radius
    ar = jnp.arange(N, dtype=jnp.int32)
    vals = jnp.where(mask, -ar, -(N + 1))
    tv, _ = jax.lax.top_k(vals, nsample)
    idx = -tv
    idx = jnp.where(idx < N, idx, idx[..., 0:1])
    return idx


def _sa(xyz, feats, npoint, radius, nsample, ws, bs):
    fidx = _fps(xyz, npoint)
    new_xyz = _gather(xyz, fidx)
    idx = _ball_query(radius, nsample, xyz, new_xyz)
    grouped = _gather(xyz, idx) - new_xyz[:, :, None, :]
    if feats is not None:
        grouped = jnp.concatenate([grouped, _gather(feats, idx)], axis=-1)
    x = grouped
    for W, b in zip(ws, bs):
        x = jax.nn.relu(jnp.einsum('bsnc,oc->bsno', x, W) + b)
    return new_xyz, jnp.max(x, axis=2)


def _fp(ux, kx, uf, kf, ws, bs):
    d = jnp.maximum(_sqdist(ux, kx), 0.0)
    negv, nidx = jax.lax.top_k(-d, 3)
    dist = -negv
    recip = 1.0 / (dist + 1e-8)
    w = recip / jnp.sum(recip, -1, keepdims=True)
    interp = jnp.sum(_gather(kf, nidx) * w[..., None], axis=2)
    x = interp if uf is None else jnp.concatenate([interp, uf], axis=-1)
    for W, b in zip(ws, bs):
        x = jax.nn.relu(jnp.einsum('bnc,oc->bno', x, W) + b)
    return x


def kernel(pointcloud, sa0_w0, sa0_b0, sa0_w1, sa0_b1, sa0_w2, sa0_b2, sa1_w0, sa1_b0, sa1_w1, sa1_b1, sa1_w2, sa1_b2, sa2_w0, sa2_b0, sa2_w1, sa2_b1, sa2_w2, sa2_b2, sa3_w0, sa3_b0, sa3_w1, sa3_b1, sa3_w2, sa3_b2, fp0_w0, fp0_b0, fp0_w1, fp0_b1, fp0_w2, fp0_b2, fp1_w0, fp1_b0, fp1_w1, fp1_b1, fp2_w0, fp2_b0, fp2_w1, fp2_b1, fp3_w0, fp3_b0, fp3_w1, fp3_b1):
    p = dict(locals())
    del p['pointcloud']
    xyz = pointcloud[..., 0:3]
    l_xyz = [xyz]
    l_f = [None]
    for L in range(4):
        ws = [p['sa%d_w%d' % (L, j)] for j in range(len(_SA_CH[L]) - 1)]
        bs = [p['sa%d_b%d' % (L, j)] for j in range(len(_SA_CH[L]) - 1)]
        nx, nf = _sa(l_xyz[L], l_f[L], _NPOINTS[L], _RADII[L], _NSAMPLE, ws, bs)
        l_xyz.append(nx)
        l_f.append(nf)
    for i in range(-1, -5, -1):
        L = 4 + i
        ws = [p['fp%d_w%d' % (L, j)] for j in range(len(_FP_CH[L]) - 1)]
        bs = [p['fp%d_b%d' % (L, j)] for j in range(len(_FP_CH[L]) - 1)]
        l_f[i - 1] = _fp(l_xyz[i - 1], l_xyz[i], l_f[i - 1], l_f[i], ws, bs)
    return jnp.transpose(l_f[0], (0, 2, 1))


# speedup vs baseline: 1.6613x; 1.4143x over previous
import jax
import jax.numpy as jnp
from jax.experimental import pallas as pl

_NPOINTS = [2048, 512, 128, 32]
_RADII = [0.1, 0.2, 0.4, 0.8]
_NSAMPLE = 32
_SA_CH = [[3, 32, 32, 64], [67, 64, 64, 128], [131, 128, 128, 256], [259, 256, 256, 512]]
_FP_CH = [[128, 128, 128, 128], [320, 256, 128], [384, 256, 256], [768, 256, 256]]


def _sqdist(a, b):
    return jnp.sum(a * a, -1)[:, :, None] + jnp.sum(b * b, -1)[:, None, :] - 2.0 * jnp.einsum('bnc,bmc->bnm', a, b)


def _gather(x, idx):
    return jax.vmap(lambda a, i: a[i])(x, idx)


def _fps(xyz, npoint):
    B, N, _ = xyz.shape
    C = 128
    R = N // C
    xs = jnp.transpose(xyz, (0, 2, 1)).reshape(B, 3, R, C)

    def kern(x_ref, o_ref):
        rr = jax.lax.broadcasted_iota(jnp.int32, (R, C), 0)
        cc = jax.lax.broadcasted_iota(jnp.int32, (R, C), 1)
        flat = rr * C + cc
        oio = jax.lax.broadcasted_iota(jnp.int32, (1, npoint), 1)
        xb = []
        for b in range(B):
            o_ref[b] = jnp.zeros((1, npoint), jnp.int32)
            xb.append((x_ref[b, 0], x_ref[b, 1], x_ref[b, 2]))

        def body(k, carry):
            new = []
            for b in range(B):
                dist, last = carry[b]
                x, y, z = xb[b]
                oh = (flat == last).astype(jnp.float32)
                lx = jnp.sum(x * oh)
                ly = jnp.sum(y * oh)
                lz = jnp.sum(z * oh)
                d = (x - lx) ** 2 + (y - ly) ** 2 + (z - lz) ** 2
                dist = jnp.minimum(dist, d)
                m = jnp.max(dist)
                nxt = jnp.min(jnp.where(dist == m, flat, N)).astype(jnp.int32)
                o_ref[b] = jnp.where(oio == k, nxt, o_ref[b])
                new.append((dist, nxt))
            return tuple(new)

        jax.lax.fori_loop(
            1, npoint, body,
            tuple((jnp.full((R, C), 1e10, jnp.float32), jnp.int32(0))
                  for _ in range(B)))

    out = pl.pallas_call(
        kern,
        in_specs=[pl.BlockSpec((B, 3, R, C), lambda: (0, 0, 0, 0))],
        out_specs=pl.BlockSpec((B, 1, npoint), lambda: (0, 0, 0)),
        out_shape=jax.ShapeDtypeStruct((B, 1, npoint), jnp.int32),
    )(xs)
    return out[:, 0, :]


def _ball_query(radius, nsample, xyz, new_xyz):
    B, N, _ = xyz.shape
    d = _sqdist(new_xyz, xyz)
    mask = d < radius * radius
    ar = jnp.arange(N, dtype=jnp.int32)
    vals = jnp.where(mask, -ar, -(N + 1))
    tv, _ = jax.lax.top_k(vals, nsample)
    idx = -tv
    idx = jnp.where(idx < N, idx, idx[..., 0:1])
    return idx


def _sa(xyz, feats, npoint, radius, nsample, ws, bs):
    fidx = _fps(xyz, npoint)
    new_xyz = _gather(xyz, fidx)
    idx = _ball_query(radius, nsample, xyz, new_xyz)
    grouped = _gather(xyz, idx) - new_xyz[:, :, None, :]
    if feats is not None:
        grouped = jnp.concatenate([grouped, _gather(feats, idx)], axis=-1)
    x = grouped
    for W, b in zip(ws, bs):
        x = jax.nn.relu(jnp.einsum('bsnc,oc->bsno', x, W) + b)
    return new_xyz, jnp.max(x, axis=2)


def _knn3(ux, kx):
    B, Nu, _ = ux.shape
    Nk = kx.shape[1]
    Sblk = min(128, Nu)
    kt = jnp.transpose(kx, (0, 2, 1))

    def kern(u_ref, k_ref, oi_ref, ow_ref):
        u = u_ref[0]
        cx = u[:, 0:1]
        cy = u[:, 1:2]
        cz = u[:, 2:3]
        x = k_ref[0, 0:1, :]
        y = k_ref[0, 1:2, :]
        z = k_ref[0, 2:3, :]
        u2 = cx * cx + cy * cy + cz * cz
        k2 = x * x + y * y + z * z
        cross = cx * x + cy * y + cz * z
        d = jnp.maximum((u2 + k2) - 2.0 * cross, 0.0)
        io = jax.lax.broadcasted_iota(jnp.int32, (Sblk, Nk), 1)
        big = jnp.float32(3.4e38)
        idxs = []
        vals = []
        for _ in range(3):
            cur = jnp.min(d, axis=1, keepdims=True)
            ci = jnp.min(jnp.where(d == cur, io, Nk), axis=1, keepdims=True)
            d = jnp.where(io == ci, big, d)
            idxs.append(ci)
            vals.append(cur)
        dist = jnp.concatenate(vals, axis=1)
        nidx = jnp.concatenate(idxs, axis=1)
        recip = 1.0 / (dist + 1e-8)
        w = recip / jnp.sum(recip, axis=1, keepdims=True)
        oi_ref[0] = nidx
        ow_ref[0] = w

    nidx, w = pl.pallas_call(
        kern,
        grid=(B, Nu // Sblk),
        in_specs=[
            pl.BlockSpec((1, Sblk, 3), lambda b, s: (b, s, 0)),
            pl.BlockSpec((1, 3, Nk), lambda b, s: (b, 0, 0)),
        ],
        out_specs=[
            pl.BlockSpec((1, Sblk, 3), lambda b, s: (b, s, 0)),
            pl.BlockSpec((1, Sblk, 3), lambda b, s: (b, s, 0)),
        ],
        out_shape=[
            jax.ShapeDtypeStruct((B, Nu, 3), jnp.int32),
            jax.ShapeDtypeStruct((B, Nu, 3), jnp.float32),
        ],
    )(ux, kt)
    return nidx, w


def _fp(ux, kx, uf, kf, ws, bs):
    nidx, w = _knn3(ux, kx)
    interp = jnp.sum(_gather(kf, nidx) * w[..., None], axis=2)
    x = interp if uf is None else jnp.concatenate([interp, uf], axis=-1)
    for W, b in zip(ws, bs):
        x = jax.nn.relu(jnp.einsum('bnc,oc->bno', x, W) + b)
    return x


def kernel(pointcloud, sa0_w0, sa0_b0, sa0_w1, sa0_b1, sa0_w2, sa0_b2, sa1_w0, sa1_b0, sa1_w1, sa1_b1, sa1_w2, sa1_b2, sa2_w0, sa2_b0, sa2_w1, sa2_b1, sa2_w2, sa2_b2, sa3_w0, sa3_b0, sa3_w1, sa3_b1, sa3_w2, sa3_b2, fp0_w0, fp0_b0, fp0_w1, fp0_b1, fp0_w2, fp0_b2, fp1_w0, fp1_b0, fp1_w1, fp1_b1, fp2_w0, fp2_b0, fp2_w1, fp2_b1, fp3_w0, fp3_b0, fp3_w1, fp3_b1):
    p = dict(locals())
    del p['pointcloud']
    xyz = pointcloud[..., 0:3]
    l_xyz = [xyz]
    l_f = [None]
    for L in range(4):
        ws = [p['sa%d_w%d' % (L, j)] for j in range(len(_SA_CH[L]) - 1)]
        bs = [p['sa%d_b%d' % (L, j)] for j in range(len(_SA_CH[L]) - 1)]
        nx, nf = _sa(l_xyz[L], l_f[L], _NPOINTS[L], _RADII[L], _NSAMPLE, ws, bs)
        l_xyz.append(nx)
        l_f.append(nf)
    for i in range(-1, -5, -1):
        L = 4 + i
        ws = [p['fp%d_w%d' % (L, j)] for j in range(len(_FP_CH[L]) - 1)]
        bs = [p['fp%d_b%d' % (L, j)] for j in range(len(_FP_CH[L]) - 1)]
        l_f[i - 1] = _fp(l_xyz[i - 1], l_xyz[i], l_f[i - 1], l_f[i], ws, bs)
    return jnp.transpose(l_f[0], (0, 2, 1))


# speedup vs baseline: 3.2165x; 1.9361x over previous
import jax
import jax.numpy as jnp
from jax.experimental import pallas as pl

_NPOINTS = [2048, 512, 128, 32]
_RADII = [0.1, 0.2, 0.4, 0.8]
_NSAMPLE = 32
_SA_CH = [[3, 32, 32, 64], [67, 64, 64, 128], [131, 128, 128, 256], [259, 256, 256, 512]]
_FP_CH = [[128, 128, 128, 128], [320, 256, 128], [384, 256, 256], [768, 256, 256]]


def _sqdist(a, b):
    return jnp.sum(a * a, -1)[:, :, None] + jnp.sum(b * b, -1)[:, None, :] - 2.0 * jnp.einsum('bnc,bmc->bnm', a, b)


def _gather(x, idx):
    return jax.vmap(lambda a, i: a[i])(x, idx)


def _fps(xyz, npoint):
    B, N, _ = xyz.shape
    C = 128
    R = N // C
    xs = jnp.transpose(xyz, (0, 2, 1)).reshape(B, 3, R, C)

    def kern(x_ref, o_ref):
        rr = jax.lax.broadcasted_iota(jnp.int32, (R, C), 0)
        cc = jax.lax.broadcasted_iota(jnp.int32, (R, C), 1)
        flat = rr * C + cc
        oio = jax.lax.broadcasted_iota(jnp.int32, (1, npoint), 1)
        xb = []
        for b in range(B):
            o_ref[b] = jnp.zeros((1, npoint), jnp.int32)
            xb.append((x_ref[b, 0], x_ref[b, 1], x_ref[b, 2]))

        def body(k, carry):
            new = []
            for b in range(B):
                dist, last = carry[b]
                x, y, z = xb[b]
                oh = (flat == last).astype(jnp.float32)
                lx = jnp.sum(x * oh)
                ly = jnp.sum(y * oh)
                lz = jnp.sum(z * oh)
                d = (x - lx) ** 2 + (y - ly) ** 2 + (z - lz) ** 2
                dist = jnp.minimum(dist, d)
                m = jnp.max(dist)
                nxt = jnp.min(jnp.where(dist == m, flat, N)).astype(jnp.int32)
                o_ref[b] = jnp.where(oio == k, nxt, o_ref[b])
                new.append((dist, nxt))
            return tuple(new)

        jax.lax.fori_loop(
            1, npoint, body,
            tuple((jnp.full((R, C), 1e10, jnp.float32), jnp.int32(0))
                  for _ in range(B)))

    out = pl.pallas_call(
        kern,
        in_specs=[pl.BlockSpec((B, 3, R, C), lambda: (0, 0, 0, 0))],
        out_specs=pl.BlockSpec((B, 1, npoint), lambda: (0, 0, 0)),
        out_shape=jax.ShapeDtypeStruct((B, 1, npoint), jnp.int32),
    )(xs)
    return out[:, 0, :]


def _ball_query(radius, nsample, xyz, new_xyz):
    B, N, _ = xyz.shape
    S = new_xyz.shape[1]
    Sblk = min(128, S)
    xt = jnp.transpose(xyz, (0, 2, 1))
    r2 = float(radius) * float(radius)

    def kern(c_ref, k_ref, o_ref):
        c = c_ref[0]
        cx = c[:, 0:1]
        cy = c[:, 1:2]
        cz = c[:, 2:3]
        x = k_ref[0, 0:1, :]
        y = k_ref[0, 1:2, :]
        z = k_ref[0, 2:3, :]
        c2 = cx * cx + cy * cy + cz * cz
        x2 = x * x + y * y + z * z
        cross = cx * x + cy * y + cz * z
        d = (c2 + x2) - 2.0 * cross
        io = jax.lax.broadcasted_iota(jnp.int32, (Sblk, N), 1)
        work = jnp.where(d < r2, io, N)
        cols = []
        for _ in range(nsample):
            cur = jnp.min(work, axis=1, keepdims=True)
            work = jnp.where(work == cur, N, work)
            cols.append(cur)
        out = jnp.concatenate(cols, axis=1)
        out = jnp.where(out < N, out, out[:, 0:1])
        o_ref[0] = out

    return pl.pallas_call(
        kern,
        grid=(B, S // Sblk),
        in_specs=[
            pl.BlockSpec((1, Sblk, 3), lambda b, s: (b, s, 0)),
            pl.BlockSpec((1, 3, N), lambda b, s: (b, 0, 0)),
        ],
        out_specs=pl.BlockSpec((1, Sblk, nsample), lambda b, s: (b, s, 0)),
        out_shape=jax.ShapeDtypeStruct((B, S, nsample), jnp.int32),
    )(new_xyz, xt)


def _sa(xyz, feats, npoint, radius, nsample, ws, bs):
    fidx = _fps(xyz, npoint)
    new_xyz = _gather(xyz, fidx)
    idx = _ball_query(radius, nsample, xyz, new_xyz)
    grouped = _gather(xyz, idx) - new_xyz[:, :, None, :]
    if feats is not None:
        grouped = jnp.concatenate([grouped, _gather(feats, idx)], axis=-1)
    x = grouped
    for W, b in zip(ws, bs):
        x = jax.nn.relu(jnp.einsum('bsnc,oc->bsno', x, W) + b)
    return new_xyz, jnp.max(x, axis=2)


def _knn3(ux, kx):
    B, Nu, _ = ux.shape
    Nk = kx.shape[1]
    Sblk = min(128, Nu)
    kt = jnp.transpose(kx, (0, 2, 1))

    def kern(u_ref, k_ref, oi_ref, ow_ref):
        u = u_ref[0]
        cx = u[:, 0:1]
        cy = u[:, 1:2]
        cz = u[:, 2:3]
        x = k_ref[0, 0:1, :]
        y = k_ref[0, 1:2, :]
        z = k_ref[0, 2:3, :]
        u2 = cx * cx + cy * cy + cz * cz
        k2 = x * x + y * y + z * z
        cross = cx * x + cy * y + cz * z
        d = jnp.maximum((u2 + k2) - 2.0 * cross, 0.0)
        io = jax.lax.broadcasted_iota(jnp.int32, (Sblk, Nk), 1)
        big = jnp.float32(3.4e38)
        idxs = []
        vals = []
        for _ in range(3):
            cur = jnp.min(d, axis=1, keepdims=True)
            ci = jnp.min(jnp.where(d == cur, io, Nk), axis=1, keepdims=True)
            d = jnp.where(io == ci, big, d)
            idxs.append(ci)
            vals.append(cur)
        dist = jnp.concatenate(vals, axis=1)
        nidx = jnp.concatenate(idxs, axis=1)
        recip = 1.0 / (dist + 1e-8)
        w = recip / jnp.sum(recip, axis=1, keepdims=True)
        oi_ref[0] = nidx
        ow_ref[0] = w

    nidx, w = pl.pallas_call(
        kern,
        grid=(B, Nu // Sblk),
        in_specs=[
            pl.BlockSpec((1, Sblk, 3), lambda b, s: (b, s, 0)),
            pl.BlockSpec((1, 3, Nk), lambda b, s: (b, 0, 0)),
        ],
        out_specs=[
            pl.BlockSpec((1, Sblk, 3), lambda b, s: (b, s, 0)),
            pl.BlockSpec((1, Sblk, 3), lambda b, s: (b, s, 0)),
        ],
        out_shape=[
            jax.ShapeDtypeStruct((B, Nu, 3), jnp.int32),
            jax.ShapeDtypeStruct((B, Nu, 3), jnp.float32),
        ],
    )(ux, kt)
    return nidx, w


def _fp(ux, kx, uf, kf, ws, bs):
    nidx, w = _knn3(ux, kx)
    interp = jnp.sum(_gather(kf, nidx) * w[..., None], axis=2)
    x = interp if uf is None else jnp.concatenate([interp, uf], axis=-1)
    for W, b in zip(ws, bs):
        x = jax.nn.relu(jnp.einsum('bnc,oc->bno', x, W) + b)
    return x


def kernel(pointcloud, sa0_w0, sa0_b0, sa0_w1, sa0_b1, sa0_w2, sa0_b2, sa1_w0, sa1_b0, sa1_w1, sa1_b1, sa1_w2, sa1_b2, sa2_w0, sa2_b0, sa2_w1, sa2_b1, sa2_w2, sa2_b2, sa3_w0, sa3_b0, sa3_w1, sa3_b1, sa3_w2, sa3_b2, fp0_w0, fp0_b0, fp0_w1, fp0_b1, fp0_w2, fp0_b2, fp1_w0, fp1_b0, fp1_w1, fp1_b1, fp2_w0, fp2_b0, fp2_w1, fp2_b1, fp3_w0, fp3_b0, fp3_w1, fp3_b1):
    p = dict(locals())
    del p['pointcloud']
    xyz = pointcloud[..., 0:3]
    l_xyz = [xyz]
    l_f = [None]
    for L in range(4):
        ws = [p['sa%d_w%d' % (L, j)] for j in range(len(_SA_CH[L]) - 1)]
        bs = [p['sa%d_b%d' % (L, j)] for j in range(len(_SA_CH[L]) - 1)]
        nx, nf = _sa(l_xyz[L], l_f[L], _NPOINTS[L], _RADII[L], _NSAMPLE, ws, bs)
        l_xyz.append(nx)
        l_f.append(nf)
    for i in range(-1, -5, -1):
        L = 4 + i
        ws = [p['fp%d_w%d' % (L, j)] for j in range(len(_FP_CH[L]) - 1)]
        bs = [p['fp%d_b%d' % (L, j)] for j in range(len(_FP_CH[L]) - 1)]
        l_f[i - 1] = _fp(l_xyz[i - 1], l_xyz[i], l_f[i - 1], l_f[i], ws, bs)
    return jnp.transpose(l_f[0], (0, 2, 1))
